# Initial kernel scaffold; baseline (speedup 1.0000x reference)
import jax
import jax.numpy as jnp
from jax.experimental import pallas as pl


def kernel(active_indices, batch_mode, W1, b1, W2, b2, W3, b3):
    raise NotImplementedError("write your pallas kernel here")


# speedup vs baseline: 3.1441x; 3.1441x over previous
import functools

import jax
import jax.numpy as jnp
from jax import lax
from jax.experimental import pallas as pl
from jax.experimental.pallas import tpu as pltpu
from jax.experimental.pallas import tpu_sc as plsc

B = 1024
A = 32
F_SIZE = 41024
H1 = 256
H2 = 32
LANES = 16
NGROUP = B // LANES
ZERO_SLOT = F_SIZE
ROW_BUF = F_SIZE + LANES


def _sc_accumulate(W1, idx_t):
    info = plsc.get_sparse_core_info()
    nc, ns = info.num_cores, info.num_subcores
    nw = nc * ns
    rows_per_w = H1 // nw

    mesh = plsc.VectorSubcoreMesh(core_axis_name="c", subcore_axis_name="s")

    @functools.partial(
        pl.kernel,
        out_type=jax.ShapeDtypeStruct((H1, B), jnp.float32),
        mesh=mesh,
        compiler_params=pltpu.CompilerParams(
            needs_layout_passes=False, use_tc_tiling_on_sc=False),
        scratch_types=[
            pltpu.VMEM((NGROUP, A, LANES), jnp.int32),
            pltpu.VMEM((ROW_BUF,), jnp.float32),
            pltpu.VMEM((ROW_BUF,), jnp.float32),
            pltpu.VMEM((B,), jnp.float32),
            pltpu.VMEM((B,), jnp.float32),
            pltpu.SemaphoreType.DMA,
            pltpu.SemaphoreType.DMA,
            pltpu.SemaphoreType.DMA,
        ],
    )
    def sc_kernel(w1_hbm, idx_hbm, out_hbm,
                  idx_v, row0, row1, orow0, orow1, sem_idx, sem_in, sem_out):
        wid = lax.axis_index("s") * nc + lax.axis_index("c")
        h0 = wid * rows_per_w

        cp_idx = pltpu.async_copy(idx_hbm, idx_v, sem_idx)

        zeros16 = jnp.zeros((LANES,), jnp.float32)
        row0[pl.ds(F_SIZE, LANES)] = zeros16
        row1[pl.ds(F_SIZE, LANES)] = zeros16

        rows = [row0, row1]
        orows = [orow0, orow1]

        in_flight = pltpu.async_copy(
            w1_hbm.at[h0], rows[0].at[pl.ds(0, F_SIZE)], sem_in)
        cp_idx.wait()

        out_flight = [None, None]
        for r in range(rows_per_w):
            buf = rows[r % 2]
            orow = orows[r % 2]
            in_flight.wait()
            if r + 1 < rows_per_w:
                in_flight = pltpu.async_copy(
                    w1_hbm.at[h0 + r + 1],
                    rows[(r + 1) % 2].at[pl.ds(0, F_SIZE)], sem_in)
            if out_flight[r % 2] is not None:
                out_flight[r % 2].wait()

            def g_body(g, _, buf=buf, orow=orow):
                acc = jnp.zeros((LANES,), jnp.float32)
                for a in range(A):
                    acc = acc + plsc.load_gather(buf, [idx_v[g, a]])
                orow[pl.ds(g * LANES, LANES)] = acc
                return 0

            lax.fori_loop(0, NGROUP, g_body, 0)
            out_flight[r % 2] = pltpu.async_copy(
                orow, out_hbm.at[h0 + r], sem_out)

        for of in out_flight:
            if of is not None:
                of.wait()

    return sc_kernel(W1, idx_t)


def _mlp_body(acc_ref, b1_ref, w2_ref, b2_ref, w3_ref, b3_ref, out_ref):
    h1 = jnp.clip(acc_ref[:] + b1_ref[:], 0.0, 1.0)
    h2 = jnp.dot(w2_ref[:], h1, preferred_element_type=jnp.float32)
    h2 = jnp.clip(h2 + b2_ref[:], 0.0, 1.0)
    o = jnp.dot(w3_ref[:], h2, preferred_element_type=jnp.float32) + b3_ref[:]
    out_ref[:] = jnp.tanh(o)


def _mlp(accT, b1, W2, b2, W3, b3):
    return pl.pallas_call(
        _mlp_body,
        out_shape=jax.ShapeDtypeStruct((1, B), jnp.float32),
    )(accT, b1.reshape(H1, 1), W2, b2.reshape(H2, 1), W3, b3.reshape(1, 1))


@jax.jit
def kernel(active_indices, batch_mode, W1, b1, W2, b2, W3, b3):
    idx = active_indices
    eq = idx[:, :, None] == idx[:, None, :]
    earlier = jnp.tril(jnp.ones((A, A), jnp.bool_), k=-1)
    is_dup = jnp.any(eq & earlier[None], axis=-1)
    dead = is_dup | (idx < 0)
    idx_f = jnp.where(dead, ZERO_SLOT, idx).astype(jnp.int32)
    idx_t = idx_f.reshape(NGROUP, LANES, A).transpose(0, 2, 1)

    accT = _sc_accumulate(W1, idx_t)
    out = _mlp(accT, b1, W2, b2, W3, b3)
    return out.reshape(B)


# speedup vs baseline: 3.2940x; 1.0477x over previous
import functools

import jax
import jax.numpy as jnp
from jax import lax
from jax.experimental import pallas as pl
from jax.experimental.pallas import tpu as pltpu
from jax.experimental.pallas import tpu_sc as plsc

B = 1024
A = 32
F_SIZE = 41024
H1 = 256
H2 = 32
LANES = 16
NGROUP = B // LANES
ZERO_SLOT = F_SIZE
ROW_BUF = F_SIZE + LANES


def _sc_accumulate(W1, idx_t):
    info = plsc.get_sparse_core_info()
    nc, ns = info.num_cores, info.num_subcores
    nw = nc * ns
    rows_per_w = H1 // nw

    mesh = plsc.VectorSubcoreMesh(core_axis_name="c", subcore_axis_name="s")

    @functools.partial(
        pl.kernel,
        out_type=jax.ShapeDtypeStruct((H1, B), jnp.float32),
        mesh=mesh,
        compiler_params=pltpu.CompilerParams(
            needs_layout_passes=False, use_tc_tiling_on_sc=False),
        scratch_types=[
            pltpu.VMEM((NGROUP, A, LANES), jnp.int32),
            pltpu.VMEM((ROW_BUF,), jnp.float32),
            pltpu.VMEM((ROW_BUF,), jnp.float32),
            pltpu.VMEM((B,), jnp.float32),
            pltpu.VMEM((B,), jnp.float32),
            pltpu.SemaphoreType.DMA,
            pltpu.SemaphoreType.DMA,
            pltpu.SemaphoreType.DMA,
        ],
    )
    def sc_kernel(w1_hbm, idx_hbm, out_hbm,
                  idx_v, row0, row1, orow0, orow1, sem_idx, sem_in, sem_out):
        wid = lax.axis_index("s") * nc + lax.axis_index("c")
        h0 = wid * rows_per_w

        cp_idx = pltpu.async_copy(idx_hbm, idx_v, sem_idx)

        zeros16 = jnp.zeros((LANES,), jnp.float32)
        row0[pl.ds(F_SIZE, LANES)] = zeros16
        row1[pl.ds(F_SIZE, LANES)] = zeros16

        rows = [row0, row1]
        orows = [orow0, orow1]

        in_flight = pltpu.async_copy(
            w1_hbm.at[h0], rows[0].at[pl.ds(0, F_SIZE)], sem_in)
        cp_idx.wait()

        out_flight = [None, None]
        for r in range(rows_per_w):
            buf = rows[r % 2]
            orow = orows[r % 2]
            in_flight.wait()
            if r + 1 < rows_per_w:
                in_flight = pltpu.async_copy(
                    w1_hbm.at[h0 + r + 1],
                    rows[(r + 1) % 2].at[pl.ds(0, F_SIZE)], sem_in)
            if out_flight[r % 2] is not None:
                out_flight[r % 2].wait()

            @plsc.parallel_loop(0, NGROUP)
            def g_body(g, buf=buf, orow=orow):
                accs = [jnp.zeros((LANES,), jnp.float32) for _ in range(4)]
                for a in range(A):
                    accs[a % 4] = accs[a % 4] + plsc.load_gather(
                        buf, [idx_v[g, a]])
                orow[pl.ds(g * LANES, LANES)] = (
                    (accs[0] + accs[1]) + (accs[2] + accs[3]))

            out_flight[r % 2] = pltpu.async_copy(
                orow, out_hbm.at[h0 + r], sem_out)

        for of in out_flight:
            if of is not None:
                of.wait()

    return sc_kernel(W1, idx_t)


def _mlp_body(acc_ref, b1_ref, w2_ref, b2_ref, w3_ref, b3_ref, out_ref):
    h1 = jnp.clip(acc_ref[:] + b1_ref[:], 0.0, 1.0)
    h2 = jnp.dot(w2_ref[:], h1, preferred_element_type=jnp.float32)
    h2 = jnp.clip(h2 + b2_ref[:], 0.0, 1.0)
    o = jnp.dot(w3_ref[:], h2, preferred_element_type=jnp.float32) + b3_ref[:]
    out_ref[:] = jnp.tanh(o)


def _mlp(accT, b1, W2, b2, W3, b3):
    return pl.pallas_call(
        _mlp_body,
        out_shape=jax.ShapeDtypeStruct((1, B), jnp.float32),
    )(accT, b1.reshape(H1, 1), W2, b2.reshape(H2, 1), W3, b3.reshape(1, 1))


@jax.jit
def kernel(active_indices, batch_mode, W1, b1, W2, b2, W3, b3):
    idx = active_indices
    eq = idx[:, :, None] == idx[:, None, :]
    earlier = jnp.tril(jnp.ones((A, A), jnp.bool_), k=-1)
    is_dup = jnp.any(eq & earlier[None], axis=-1)
    dead = is_dup | (idx < 0)
    idx_f = jnp.where(dead, ZERO_SLOT, idx).astype(jnp.int32)
    idx_t = idx_f.reshape(NGROUP, LANES, A).transpose(0, 2, 1)
    accT = _sc_accumulate(W1, idx_t)
    out = _mlp(accT, b1, W2, b2, W3, b3)
    return out.reshape(B)
